# Initial kernel scaffold; baseline (speedup 1.0000x reference)
import functools

import jax
import jax.numpy as jnp
from jax import lax
from jax.experimental import pallas as pl
from jax.experimental.pallas import tpu as pltpu
from jax.experimental.pallas import tpu_sc as plsc

_N = 10000
_E = 320000
_D = 128

_K = 128
_NCHUNKS = _E // _K
_NC = 2
_NS = 16
_NT = _NC * _NS
_RPT = _N // _NS

_mesh = plsc.VectorSubcoreMesh(core_axis_name="c", subcore_axis_name="s")


@functools.partial(
    pl.kernel,
    out_type=jax.ShapeDtypeStruct((_NC, _N), jnp.float32),
    mesh=_mesh,
    scratch_types=[
        pltpu.VMEM((1, _K), jnp.int32),
        pltpu.VMEM((_K,), jnp.float32),
        pltpu.VMEM((2000,), jnp.float32),
        pltpu.VMEM_SHARED((_N,), jnp.float32),
    ],
)
def _sc_count(dst_hbm, out_hbm, dst_v, ones_v, zbuf, acc):
    c = lax.axis_index("c")
    s = lax.axis_index("s")
    t = c * _NS + s

    def setv(i, _):
        ones_v[pl.ds(i * 16, 16)] = jnp.ones((16,), jnp.float32)
        return 0

    lax.fori_loop(0, _K // 16, setv, 0)

    @pl.when(s == 0)
    def _zero():
        def zr(i, _):
            zbuf[pl.ds(i * 16, 16)] = jnp.zeros((16,), jnp.float32)
            return 0

        lax.fori_loop(0, 125, zr, 0)
        for i in range(5):
            pltpu.sync_copy(zbuf, acc.at[pl.ds(i * 2000, 2000)])

    plsc.subcore_barrier()

    g0 = (t * _NCHUNKS) // _NT
    g1 = ((t + 1) * _NCHUNKS) // _NT

    def body(g, _):
        pltpu.sync_copy(dst_hbm.at[pl.ds(g * _K, _K)], dst_v.at[0])
        pltpu.sync_copy(ones_v, acc.at[dst_v.at[0]], add=True)
        return 0

    lax.fori_loop(g0, g1, body, 0)
    plsc.subcore_barrier()

    @pl.when(s == 0)
    def _out():
        pltpu.sync_copy(acc, out_hbm.at[c])


@functools.partial(
    pl.kernel,
    out_type=jax.ShapeDtypeStruct((_NC, _N, _D), jnp.float32),
    mesh=_mesh,
    scratch_types=[
        pltpu.VMEM((_K,), jnp.int32),
        pltpu.VMEM((1, _K), jnp.int32),
        pltpu.VMEM((_K, _D), jnp.float32),
        pltpu.VMEM((125, _D), jnp.float32),
        pltpu.VMEM_SHARED((_N, _D), jnp.float32),
        pltpu.SemaphoreType.DMA,
    ],
)
def _sc_agg(y_hbm, src_hbm, dst_hbm, out_hbm, src_v, dst_v, rows_v, zbuf, acc, sem):
    c = lax.axis_index("c")
    s = lax.axis_index("s")
    t = c * _NS + s

    def zrow(i, _):
        for j in range(_D // 16):
            zbuf[i, pl.ds(j * 16, 16)] = jnp.zeros((16,), jnp.float32)
        return 0

    lax.fori_loop(0, 125, zrow, 0)
    base = s * _RPT
    for i in range(_RPT // 125):
        pltpu.sync_copy(zbuf, acc.at[pl.ds(base + i * 125, 125)])

    plsc.subcore_barrier()

    g0 = (t * _NCHUNKS) // _NT
    g1 = ((t + 1) * _NCHUNKS) // _NT

    def body(g, _):
        e0 = g * _K
        pltpu.sync_copy(src_hbm.at[pl.ds(e0, _K)], src_v)
        pltpu.sync_copy(dst_hbm.at[pl.ds(e0, _K)], dst_v.at[0])
        pltpu.async_copy(y_hbm.at[src_v], rows_v, sem).wait()
        pltpu.sync_copy(rows_v, acc.at[dst_v.at[0]], add=True)
        return 0

    lax.fori_loop(g0, g1, body, 0)
    plsc.subcore_barrier()
    pltpu.sync_copy(acc.at[pl.ds(base, _RPT)], out_hbm.at[c, pl.ds(base, _RPT)])


def _tc_prep_body(x_ref, w_ref, cnt_ref, y_ref, dinv_ref):
    deg = cnt_ref[0] + cnt_ref[1] + 1.0
    dinv = lax.rsqrt(deg)
    h = jnp.dot(x_ref[...], w_ref[...], preferred_element_type=jnp.float32)
    y_ref[...] = h * dinv
    dinv_ref[...] = dinv


_tc_prep = pl.pallas_call(
    _tc_prep_body,
    out_shape=(
        jax.ShapeDtypeStruct((_N, _D), jnp.float32),
        jax.ShapeDtypeStruct((_N, 1), jnp.float32),
    ),
)


def _tc_mid_body(p_ref, y1_ref, dinv_ref, b_ref, w_ref, y2_ref):
    agg = p_ref[0] + p_ref[1] + y1_ref[...]
    out1 = dinv_ref[...] * agg + b_ref[...]
    h2 = jnp.dot(out1, w_ref[...], preferred_element_type=jnp.float32)
    y2_ref[...] = h2 * dinv_ref[...]


_tc_mid = pl.pallas_call(
    _tc_mid_body,
    out_shape=jax.ShapeDtypeStruct((_N, _D), jnp.float32),
)


def _tc_fin_body(q_ref, y2_ref, dinv_ref, b_ref, o_ref):
    z = dinv_ref[...] * (q_ref[0] + q_ref[1] + y2_ref[...]) + b_ref[...]
    o_ref[...] = jnp.where(z >= 0, z, 0.1 * z)


_tc_fin = pl.pallas_call(
    _tc_fin_body,
    out_shape=jax.ShapeDtypeStruct((_N, _D), jnp.float32),
)


@jax.jit
def _run(x, edge_index, W1, b1, W2, b2):
    src = edge_index[0]
    dst = edge_index[1]
    cnt = _sc_count(dst).reshape(_NC, _N, 1)
    y1, dinv = _tc_prep(x, W1, cnt)
    p = _sc_agg(y1, src, dst)
    y2 = _tc_mid(p, y1, dinv, b1.reshape(1, _D), W2)
    q = _sc_agg(y2, src, dst)
    return _tc_fin(q, y2, dinv, b2.reshape(1, _D))


def kernel(x, edge_index, W1, b1, W2, b2):
    return _run(x, edge_index, W1, b1, W2, b2)


# speedup vs baseline: 16.5669x; 16.5669x over previous
import functools

import jax
import jax.numpy as jnp
from jax import lax
from jax.experimental import pallas as pl
from jax.experimental.pallas import tpu as pltpu
from jax.experimental.pallas import tpu_sc as plsc

_N = 10000
_E = 320000
_D = 128

_K = 128
_NCHUNKS = _E // _K
_NC = 2
_NS = 16
_NT = _NC * _NS
_NP = 10240
_RPT = _NP // _NS

_mesh = plsc.VectorSubcoreMesh(core_axis_name="c", subcore_axis_name="s")


@functools.partial(
    pl.kernel,
    out_type=jax.ShapeDtypeStruct((_NC, _NP), jnp.float32),
    mesh=_mesh,
    scratch_types=[
        pltpu.VMEM((1, _K), jnp.int32),
        pltpu.VMEM((_K,), jnp.float32),
        pltpu.VMEM((2048,), jnp.float32),
        pltpu.VMEM_SHARED((_NP,), jnp.float32),
    ],
)
def _sc_count(dst_hbm, out_hbm, dst_v, ones_v, zbuf, acc):
    c = lax.axis_index("c")
    s = lax.axis_index("s")
    t = c * _NS + s

    def setv(i, _):
        ones_v[pl.ds(i * 16, 16)] = jnp.ones((16,), jnp.float32)
        return 0

    lax.fori_loop(0, _K // 16, setv, 0)

    @pl.when(s == 0)
    def _zero():
        def zr(i, _):
            zbuf[pl.ds(i * 16, 16)] = jnp.zeros((16,), jnp.float32)
            return 0

        lax.fori_loop(0, 2048 // 16, zr, 0)
        for i in range(_NP // 2048):
            pltpu.sync_copy(zbuf, acc.at[pl.ds(i * 2048, 2048)])

    plsc.subcore_barrier()

    g0 = (t * _NCHUNKS) // _NT
    g1 = ((t + 1) * _NCHUNKS) // _NT

    def body(g, _):
        pltpu.sync_copy(dst_hbm.at[pl.ds(g * _K, _K)], dst_v.at[0])
        pltpu.sync_copy(ones_v, acc.at[dst_v.at[0]], add=True)
        return 0

    lax.fori_loop(g0, g1, body, 0)
    plsc.subcore_barrier()

    @pl.when(s == 0)
    def _out():
        pltpu.sync_copy(acc, out_hbm.at[c])


@functools.partial(
    pl.kernel,
    out_type=jax.ShapeDtypeStruct((_NC, _NP, _D), jnp.float32),
    mesh=_mesh,
    scratch_types=[
        pltpu.VMEM((_K,), jnp.int32),
        pltpu.VMEM((1, _K), jnp.int32),
        pltpu.VMEM((_K, _D), jnp.float32),
        pltpu.VMEM((128, _D), jnp.float32),
        pltpu.VMEM_SHARED((_NP, _D), jnp.float32),
        pltpu.SemaphoreType.DMA,
    ],
)
def _sc_agg(y_hbm, src_hbm, dst_hbm, out_hbm, src_v, dst_v, rows_v, zbuf, acc, sem):
    c = lax.axis_index("c")
    s = lax.axis_index("s")
    t = c * _NS + s

    def zrow(i, _):
        for j in range(_D // 16):
            zbuf[i, pl.ds(j * 16, 16)] = jnp.zeros((16,), jnp.float32)
        return 0

    lax.fori_loop(0, 128, zrow, 0)
    base = s * _RPT
    for i in range(_RPT // 128):
        pltpu.sync_copy(zbuf, acc.at[pl.ds(base + i * 128, 128)])

    plsc.subcore_barrier()

    g0 = (t * _NCHUNKS) // _NT
    g1 = ((t + 1) * _NCHUNKS) // _NT

    def body(g, _):
        e0 = g * _K
        pltpu.sync_copy(src_hbm.at[pl.ds(e0, _K)], src_v)
        pltpu.sync_copy(dst_hbm.at[pl.ds(e0, _K)], dst_v.at[0])
        pltpu.async_copy(y_hbm.at[src_v], rows_v, sem).wait()
        pltpu.sync_copy(rows_v, acc.at[dst_v.at[0]], add=True)
        return 0

    lax.fori_loop(g0, g1, body, 0)
    plsc.subcore_barrier()
    pltpu.sync_copy(acc.at[pl.ds(base, _RPT)], out_hbm.at[c, pl.ds(base, _RPT)])


def _tc_prep_body(x_ref, w_ref, cnt_ref, y_ref, dinv_ref):
    deg = cnt_ref[0] + cnt_ref[1] + 1.0
    dinv = lax.rsqrt(deg)
    h = jnp.dot(x_ref[...], w_ref[...], preferred_element_type=jnp.float32)
    y_ref[...] = h * dinv
    dinv_ref[...] = dinv


_tc_prep = pl.pallas_call(
    _tc_prep_body,
    out_shape=(
        jax.ShapeDtypeStruct((_NP, _D), jnp.float32),
        jax.ShapeDtypeStruct((_NP, 1), jnp.float32),
    ),
)


def _tc_mid_body(p_ref, y1_ref, dinv_ref, b_ref, w_ref, y2_ref):
    agg = p_ref[0] + p_ref[1] + y1_ref[...]
    out1 = dinv_ref[...] * agg + b_ref[...]
    h2 = jnp.dot(out1, w_ref[...], preferred_element_type=jnp.float32)
    y2_ref[...] = h2 * dinv_ref[...]


_tc_mid = pl.pallas_call(
    _tc_mid_body,
    out_shape=jax.ShapeDtypeStruct((_NP, _D), jnp.float32),
)


def _tc_fin_body(q_ref, y2_ref, dinv_ref, b_ref, o_ref):
    z = dinv_ref[...] * (q_ref[0] + q_ref[1] + y2_ref[...]) + b_ref[...]
    o_ref[...] = jnp.where(z >= 0, z, 0.1 * z)


_tc_fin = pl.pallas_call(
    _tc_fin_body,
    out_shape=jax.ShapeDtypeStruct((_NP, _D), jnp.float32),
)


@jax.jit
def _run(x, edge_index, W1, b1, W2, b2):
    src = edge_index[0]
    dst = edge_index[1]
    xp = jnp.pad(x, ((0, _NP - _N), (0, 0)))
    cnt = _sc_count(dst).reshape(_NC, _NP, 1)
    y1, dinv = _tc_prep(xp, W1, cnt)
    p = _sc_agg(y1, src, dst)
    y2 = _tc_mid(p, y1, dinv, b1.reshape(1, _D), W2)
    q = _sc_agg(y2, src, dst)
    outp = _tc_fin(q, y2, dinv, b2.reshape(1, _D))
    return outp[:_N]


def kernel(x, edge_index, W1, b1, W2, b2):
    return _run(x, edge_index, W1, b1, W2, b2)


# speedup vs baseline: 35.2034x; 2.1249x over previous
import functools

import jax
import jax.numpy as jnp
from jax import lax
from jax.experimental import pallas as pl
from jax.experimental.pallas import tpu as pltpu
from jax.experimental.pallas import tpu_sc as plsc

_N = 10000
_E = 320000
_D = 128

_K = 80
_NCHUNKS = _E // _K
_NC = 2
_NS = 16
_NT = _NC * _NS
_CPT = _NCHUNKS // _NT
_EPT = _E // _NT
_GBUF = 3
_IBUF = 6
_NP = 10112
_RPT = _NP // _NS

_mesh = plsc.VectorSubcoreMesh(core_axis_name="c", subcore_axis_name="s")


@functools.partial(
    pl.kernel,
    out_type=jax.ShapeDtypeStruct((_NC, _NP), jnp.float32),
    mesh=_mesh,
    scratch_types=[
        pltpu.VMEM((4, _K), jnp.int32),
        pltpu.VMEM((_K,), jnp.float32),
        pltpu.VMEM((640,), jnp.float32),
        pltpu.VMEM_SHARED((_NP,), jnp.float32),
        pltpu.SemaphoreType.DMA,
        pltpu.SemaphoreType.DMA,
        pltpu.SemaphoreType.DMA,
        pltpu.SemaphoreType.DMA,
    ],
)
def _sc_count(dst4_hbm, out_hbm, idx_v, ones_v, zbuf, acc, *isem):
    c = lax.axis_index("c")
    s = lax.axis_index("s")
    t = c * _NS + s
    c0 = t * _CPT

    def setv(i, _):
        ones_v[pl.ds(i * 16, 16)] = jnp.ones((16,), jnp.float32)
        return 0

    lax.fori_loop(0, _K // 16, setv, 0)

    def zr(i, _):
        zbuf[pl.ds(i * 16, 16)] = jnp.zeros((16,), jnp.float32)
        return 0

    lax.fori_loop(0, 640 // 16, zr, 0)

    @pl.when(s < _NS - 1)
    def _zmain():
        pltpu.sync_copy(zbuf, acc.at[pl.ds(s * 640, 640)])

    @pl.when(s == _NS - 1)
    def _ztail():
        pltpu.sync_copy(zbuf.at[pl.ds(0, 512)], acc.at[pl.ds(9600, 512)])

    def ifetch(cl, q):
        pltpu.async_copy(dst4_hbm.at[c0 + cl], idx_v.at[pl.ds(q, 1)], isem[q])

    def iwait(cl, q):
        pltpu.make_async_copy(
            dst4_hbm.at[c0 + cl], idx_v.at[pl.ds(q, 1)], isem[q]).wait()

    for u in range(4):
        ifetch(u, u)
    plsc.subcore_barrier()

    def chunk(cl, u):
        q = u % 4
        iwait(cl, q)
        pltpu.sync_copy(ones_v, acc.at[idx_v.at[q]], add=True)
        if isinstance(cl, int):
            if cl + 4 < _CPT:
                ifetch(cl + 4, q)
        else:
            @pl.when(cl + 4 < _CPT)
            def _pf():
                ifetch(cl + 4, q)

    def body(j, _):
        for u in range(4):
            chunk(j * 4 + u, u)
        return 0

    lax.fori_loop(0, 30, body, 0)
    for cl in range(120, _CPT):
        chunk(cl, cl % 4)

    plsc.subcore_barrier()

    @pl.when(s == 0)
    def _out():
        pltpu.sync_copy(acc, out_hbm.at[c])


@functools.partial(
    pl.kernel,
    out_type=jax.ShapeDtypeStruct((_NC, _NP, _D), jnp.float32),
    mesh=_mesh,
    scratch_types=[
        pltpu.VMEM((_IBUF, _K), jnp.int32),
        pltpu.VMEM((_IBUF, _K), jnp.int32),
        pltpu.VMEM((_GBUF, _K, _D), jnp.float32),
        pltpu.VMEM_SHARED((_NP, _D), jnp.float32),
    ]
    + [pltpu.SemaphoreType.DMA] * (2 * _IBUF)
    + [pltpu.SemaphoreType.DMA] * _GBUF,
)
def _sc_agg(y_hbm, src4_hbm, dst4_hbm, out_hbm, src_v, dst_v, rows_v, acc,
            *sems):
    ssem = sems[:_IBUF]
    dsem = sems[_IBUF:2 * _IBUF]
    gsem = sems[2 * _IBUF:]
    c = lax.axis_index("c")
    s = lax.axis_index("s")
    t = c * _NS + s
    c0 = t * _CPT

    def zrow(i, _):
        for j in range(_D // 16):
            rows_v[0, i, pl.ds(j * 16, 16)] = jnp.zeros((16,), jnp.float32)
        return 0

    lax.fori_loop(0, _K, zrow, 0)
    base = s * _RPT
    for i in range(7):
        pltpu.sync_copy(rows_v.at[0], acc.at[pl.ds(base + i * _K, _K)])
    pltpu.sync_copy(rows_v.at[0, pl.ds(0, 72)], acc.at[pl.ds(base + 560, 72)])

    def ifetch(cl, q):
        pltpu.async_copy(src4_hbm.at[c0 + cl], src_v.at[pl.ds(q, 1)], ssem[q])
        pltpu.async_copy(dst4_hbm.at[c0 + cl], dst_v.at[pl.ds(q, 1)], dsem[q])

    def iwait(cl, q):
        pltpu.make_async_copy(
            src4_hbm.at[c0 + cl], src_v.at[pl.ds(q, 1)], ssem[q]).wait()
        pltpu.make_async_copy(
            dst4_hbm.at[c0 + cl], dst_v.at[pl.ds(q, 1)], dsem[q]).wait()

    def gfetch(q, r):
        pltpu.async_copy(y_hbm.at[src_v.at[q]], rows_v.at[r], gsem[r])

    def gwait(q, r):
        pltpu.make_async_copy(
            y_hbm.at[src_v.at[q]], rows_v.at[r], gsem[r]).wait()

    for u in range(_IBUF):
        ifetch(u, u)
    for u in range(_GBUF):
        iwait(u, u)
        gfetch(u, u)
    plsc.subcore_barrier()

    def chunk(cl, u):
        r = u % _GBUF
        q = u % _IBUF
        qn = (u + _GBUF) % _IBUF
        gwait(q, r)
        pltpu.sync_copy(rows_v.at[r], acc.at[dst_v.at[q]], add=True)
        if isinstance(cl, int):
            if cl + _IBUF < _CPT:
                ifetch(cl + _IBUF, q)
            if cl + _GBUF < _CPT:
                iwait(cl + _GBUF, qn)
                gfetch(qn, r)
        else:
            ifetch(cl + _IBUF, q)
            iwait(cl + _GBUF, qn)
            gfetch(qn, r)

    def body(j, _):
        for u in range(_IBUF):
            chunk(j * _IBUF + u, u)
        return 0

    lax.fori_loop(0, 19, body, 0)
    for cl in range(114, _CPT):
        chunk(cl, cl % _IBUF)

    plsc.subcore_barrier()
    pltpu.sync_copy(acc.at[pl.ds(base, _RPT)], out_hbm.at[c, pl.ds(base, _RPT)])


def _tc_prep_body(x_ref, w_ref, cnt_ref, y_ref, dinv_ref):
    deg = cnt_ref[0] + cnt_ref[1] + 1.0
    dinv = lax.rsqrt(deg)
    h = jnp.dot(x_ref[...], w_ref[...], preferred_element_type=jnp.float32)
    y_ref[...] = h * dinv
    dinv_ref[...] = dinv


_tc_prep = pl.pallas_call(
    _tc_prep_body,
    out_shape=(
        jax.ShapeDtypeStruct((_NP, _D), jnp.float32),
        jax.ShapeDtypeStruct((_NP, 1), jnp.float32),
    ),
)


def _tc_mid_body(p_ref, y1_ref, dinv_ref, b_ref, w_ref, y2_ref):
    agg = p_ref[0] + p_ref[1] + y1_ref[...]
    out1 = dinv_ref[...] * agg + b_ref[...]
    h2 = jnp.dot(out1, w_ref[...], preferred_element_type=jnp.float32)
    y2_ref[...] = h2 * dinv_ref[...]


_tc_mid = pl.pallas_call(
    _tc_mid_body,
    out_shape=jax.ShapeDtypeStruct((_NP, _D), jnp.float32),
)


def _tc_fin_body(q_ref, y2_ref, dinv_ref, b_ref, o_ref):
    z = dinv_ref[...] * (q_ref[0] + q_ref[1] + y2_ref[...]) + b_ref[...]
    o_ref[...] = jnp.where(z >= 0, z, 0.1 * z)


_tc_fin = pl.pallas_call(
    _tc_fin_body,
    out_shape=jax.ShapeDtypeStruct((_NP, _D), jnp.float32),
)


@jax.jit
def _run(x, edge_index, W1, b1, W2, b2):
    src4 = edge_index[0].reshape(_NCHUNKS, 1, _K)
    dst4 = edge_index[1].reshape(_NCHUNKS, 1, _K)
    xp = jnp.pad(x, ((0, _NP - _N), (0, 0)))
    cnt = _sc_count(dst4).reshape(_NC, _NP, 1)
    y1, dinv = _tc_prep(xp, W1, cnt)
    p = _sc_agg(y1, src4, dst4)
    y2 = _tc_mid(p, y1, dinv, b1.reshape(1, _D), W2)
    q = _sc_agg(y2, src4, dst4)
    outp = _tc_fin(q, y2, dinv, b2.reshape(1, _D))
    return outp[:_N]


def kernel(x, edge_index, W1, b1, W2, b2):
    return _run(x, edge_index, W1, b1, W2, b2)
